# baseline (speedup 1.0000x reference)
import functools

import jax
import jax.numpy as jnp
from jax import lax
from jax.experimental import pallas as pl
from jax.experimental.pallas import tpu as pltpu
from jax.experimental.pallas import tpu_sc as plsc

_B = 16384
_F = 26
_N = _B * _F
_D = 16
_V = 1000000
_V8 = _V // 8
_NW = 32
_PER_W = _N // _NW
_G = 128
_NG = _PER_W // _G


def _make_kernel():
    mesh = plsc.VectorSubcoreMesh(core_axis_name="c", subcore_axis_name="s")

    @functools.partial(
        pl.kernel,
        mesh=mesh,
        compiler_params=pltpu.CompilerParams(use_tc_tiling_on_sc=True),
        out_type=jax.ShapeDtypeStruct((_N, 2, _D), jnp.float32),
        scratch_types=[
            pltpu.VMEM((_PER_W,), jnp.int32),
            pltpu.VMEM((_PER_W,), jnp.int32),
            pltpu.VMEM((2, _G, 8 * _D), jnp.float32),
            pltpu.VMEM((2, _G, 8 * _D), jnp.float32),
            pltpu.VMEM((_G, _D), jnp.float32),
            pltpu.VMEM((_G, _D), jnp.float32),
            pltpu.SemaphoreType.DMA,
            pltpu.SemaphoreType.DMA,
        ],
    )
    def k(x_hbm, frozen_hbm, learn_hbm, out_hbm,
          idx_v, idx8_v, fs_buf, ls_buf, f_buf, l_buf, s0, s1):
        sg = [s0, s1]
        c = lax.axis_index("c")
        s = lax.axis_index("s")
        base = (s * 2 + c) * _PER_W
        pltpu.sync_copy(x_hbm.at[pl.ds(base, _PER_W)], idx_v)

        def shift(i, carry):
            sl = pl.ds(i * 16, 16)
            idx8_v[sl] = idx_v[sl] >> 3
            return carry

        lax.fori_loop(0, _PER_W // 16, shift, 0)

        def gathers(j, b):
            i8 = idx8_v.at[pl.ds(j * _G, _G)]
            pltpu.async_copy(frozen_hbm.at[i8], fs_buf.at[b], sg[b])
            pltpu.async_copy(learn_hbm.at[i8], ls_buf.at[b], sg[b])

        def wait_gathers(j, b):
            i8 = idx8_v.at[pl.ds(j * _G, _G)]
            pltpu.make_async_copy(frozen_hbm.at[i8], fs_buf.at[b], sg[b]).wait()
            pltpu.make_async_copy(learn_hbm.at[i8], ls_buf.at[b], sg[b]).wait()

        def extract_and_write(j, b):
            off = j * _G
            for g8 in range(_G // 16):
                mvec = (idx_v[pl.ds(off + g8 * 16, 16)] & 7) * 16
                for kk in range(16):
                    r = g8 * 16 + kk
                    m = mvec[kk]
                    f_buf[r, :] = fs_buf[b, r, pl.ds(m, _D)]
                    l_buf[r, :] = ls_buf[b, r, pl.ds(m, _D)]
            o = pl.ds(base + off, _G)
            pltpu.sync_copy(f_buf, out_hbm.at[o, 0])
            pltpu.sync_copy(l_buf, out_hbm.at[o, 1])

        gathers(0, 0)

        def step(g, carry):
            for b in range(2):
                j = g * 2 + b

                @pl.when(j + 1 < _NG)
                def _():
                    gathers(j + 1, 1 - b)

                wait_gathers(j, b)
                extract_and_write(j, b)
            return carry

        lax.fori_loop(0, _NG // 2, step, 0)

    return k


_sc_gather = _make_kernel()


def kernel(x, frozen_table, learn_table):
    x_flat = x.reshape(_N).astype(jnp.int32)
    f8 = frozen_table.reshape(_V8, 8 * _D)
    l8 = learn_table.reshape(_V8, 8 * _D)
    out = _sc_gather(x_flat, f8, l8)
    return out.reshape(_B, _F, 2 * _D)


# speedup vs baseline: 1.9651x; 1.9651x over previous
import functools

import jax
import jax.numpy as jnp
from jax import lax
from jax.experimental import pallas as pl
from jax.experimental.pallas import tpu as pltpu
from jax.experimental.pallas import tpu_sc as plsc

_B = 16384
_F = 26
_N = _B * _F
_D = 16
_NW = 32
_PER_W = _N // _NW
_G = 128
_NG = _PER_W // _G
_P = 4
_UNROLL = 4


def _make_kernel():
    mesh = plsc.VectorSubcoreMesh(core_axis_name="c", subcore_axis_name="s")

    @functools.partial(
        pl.kernel,
        mesh=mesh,
        compiler_params=pltpu.CompilerParams(use_tc_tiling_on_sc=False),
        out_type=jax.ShapeDtypeStruct((_N, 2, _D), jnp.float32),
        scratch_types=[
            pltpu.VMEM((_PER_W,), jnp.int32),
            pltpu.VMEM((_P, _G, _D), jnp.float32),
            pltpu.VMEM((_P, _G, _D), jnp.float32),
        ] + [pltpu.SemaphoreType.DMA] * (2 * _P),
    )
    def k(x_hbm, frozen_hbm, learn_hbm, out_hbm, idx_v, f_buf, l_buf, *sems):
        sg = sems[:_P]
        sw = sems[_P:]
        c = lax.axis_index("c")
        s = lax.axis_index("s")
        base = (s * 2 + c) * _PER_W
        pltpu.sync_copy(x_hbm.at[pl.ds(base, _PER_W)], idx_v)

        def gathers(j, b):
            idx = idx_v.at[pl.ds(j * _G, _G)]
            pltpu.async_copy(frozen_hbm.at[idx], f_buf.at[b], sg[b])
            pltpu.async_copy(learn_hbm.at[idx], l_buf.at[b], sg[b])

        def wait_gathers(j, b):
            idx = idx_v.at[pl.ds(j * _G, _G)]
            pltpu.make_async_copy(frozen_hbm.at[idx], f_buf.at[b], sg[b]).wait()
            pltpu.make_async_copy(learn_hbm.at[idx], l_buf.at[b], sg[b]).wait()

        def writes(j, b):
            o = pl.ds(base + j * _G, _G)
            pltpu.async_copy(f_buf.at[b], out_hbm.at[o, 0], sw[b])
            pltpu.async_copy(l_buf.at[b], out_hbm.at[o, 1], sw[b])

        def wait_writes(j, b):
            o = pl.ds(base + j * _G, _G)
            pltpu.make_async_copy(f_buf.at[b], out_hbm.at[o, 0], sw[b]).wait()
            pltpu.make_async_copy(l_buf.at[b], out_hbm.at[o, 1], sw[b]).wait()

        gathers(0, 0)
        gathers(1, 1)

        def step(g, carry):
            for b in range(_UNROLL):
                j = g * _UNROLL + b
                bb = b % _P
                wait_gathers(j, bb)
                writes(j, bb)
                b2 = (b + 2) % _P

                @pl.when(j >= 2)
                def _():
                    wait_writes(j - 2, b2)

                @pl.when(j + 2 < _NG)
                def _():
                    gathers(j + 2, b2)

            return carry

        lax.fori_loop(0, _NG // _UNROLL, step, 0)
        wait_writes(_NG - 2, (_NG - 2) % _P)
        wait_writes(_NG - 1, (_NG - 1) % _P)

    return k


_sc_gather = _make_kernel()


def kernel(x, frozen_table, learn_table):
    x_flat = x.reshape(_N).astype(jnp.int32)
    out = _sc_gather(x_flat, frozen_table, learn_table)
    return out.reshape(_B, _F, 2 * _D)


# speedup vs baseline: 1.9939x; 1.0146x over previous
import functools

import jax
import jax.numpy as jnp
from jax import lax
from jax.experimental import pallas as pl
from jax.experimental.pallas import tpu as pltpu
from jax.experimental.pallas import tpu_sc as plsc

_B = 16384
_F = 26
_N = _B * _F
_D = 16
_NW = 32
_PER_W = _N // _NW
_G = 256
_NG = _PER_W // _G
_P = 4
_UNROLL = 4


def _make_kernel():
    mesh = plsc.VectorSubcoreMesh(core_axis_name="c", subcore_axis_name="s")

    @functools.partial(
        pl.kernel,
        mesh=mesh,
        compiler_params=pltpu.CompilerParams(use_tc_tiling_on_sc=False),
        out_type=jax.ShapeDtypeStruct((_N, 2, _D), jnp.float32),
        scratch_types=[
            pltpu.VMEM((_PER_W,), jnp.int32),
            pltpu.VMEM((_P, _G, _D), jnp.float32),
            pltpu.VMEM((_P, _G, _D), jnp.float32),
        ] + [pltpu.SemaphoreType.DMA] * (2 * _P),
    )
    def k(x_hbm, frozen_hbm, learn_hbm, out_hbm, idx_v, f_buf, l_buf, *sems):
        sg = sems[:_P]
        sw = sems[_P:]
        c = lax.axis_index("c")
        s = lax.axis_index("s")
        base = (s * 2 + c) * _PER_W
        pltpu.sync_copy(x_hbm.at[pl.ds(base, _PER_W)], idx_v)

        def gathers(j, b):
            idx = idx_v.at[pl.ds(j * _G, _G)]
            pltpu.async_copy(frozen_hbm.at[idx], f_buf.at[b], sg[b])
            pltpu.async_copy(learn_hbm.at[idx], l_buf.at[b], sg[b])

        def wait_gathers(j, b):
            idx = idx_v.at[pl.ds(j * _G, _G)]
            pltpu.make_async_copy(frozen_hbm.at[idx], f_buf.at[b], sg[b]).wait()
            pltpu.make_async_copy(learn_hbm.at[idx], l_buf.at[b], sg[b]).wait()

        def writes(j, b):
            o = pl.ds(base + j * _G, _G)
            pltpu.async_copy(f_buf.at[b], out_hbm.at[o, 0], sw[b])
            pltpu.async_copy(l_buf.at[b], out_hbm.at[o, 1], sw[b])

        def wait_writes(j, b):
            o = pl.ds(base + j * _G, _G)
            pltpu.make_async_copy(f_buf.at[b], out_hbm.at[o, 0], sw[b]).wait()
            pltpu.make_async_copy(l_buf.at[b], out_hbm.at[o, 1], sw[b]).wait()

        gathers(0, 0)
        gathers(1, 1)

        def step(g, carry):
            for b in range(_UNROLL):
                j = g * _UNROLL + b
                bb = b % _P
                wait_gathers(j, bb)
                writes(j, bb)
                b2 = (b + 2) % _P

                @pl.when(j >= 2)
                def _():
                    wait_writes(j - 2, b2)

                @pl.when(j + 2 < _NG)
                def _():
                    gathers(j + 2, b2)

            return carry

        lax.fori_loop(0, _NG // _UNROLL, step, 0)
        wait_writes(_NG - 2, (_NG - 2) % _P)
        wait_writes(_NG - 1, (_NG - 1) % _P)

    return k


_sc_gather = _make_kernel()


def kernel(x, frozen_table, learn_table):
    x_flat = x.reshape(_N).astype(jnp.int32)
    out = _sc_gather(x_flat, frozen_table, learn_table)
    return out.reshape(_B, _F, 2 * _D)
